# Initial kernel scaffold; baseline (speedup 1.0000x reference)
import jax
import jax.numpy as jnp
from jax.experimental import pallas as pl


def kernel(inputs, pos_matrix):
    raise NotImplementedError("write your pallas kernel here")


# speedup vs baseline: 3.7255x; 3.7255x over previous
import functools

import jax
import jax.numpy as jnp
from jax import lax
from jax.experimental import pallas as pl
from jax.experimental.pallas import tpu as pltpu
from jax.experimental.pallas import tpu_sc as plsc

_INPUT_DIM = 8192
_OUTPUT_DIM = 64

_NC = 2
_NS = 16
_NW = _NC * _NS

_B = 4096 * 200
_BPW = _B // _NW
_CHUNK = 128
_NCHUNK = _BPW // _CHUNK

_mesh = plsc.VectorSubcoreMesh(core_axis_name="c", subcore_axis_name="s")


@functools.partial(
    pl.kernel,
    out_type=jax.ShapeDtypeStruct((_B, _OUTPUT_DIM), jnp.float32),
    mesh=_mesh,
    scratch_types=[
        pltpu.VMEM((_CHUNK,), jnp.int32),
        pltpu.VMEM((_CHUNK, _OUTPUT_DIM), jnp.float32),
        pltpu.SemaphoreType.DMA,
    ],
    compiler_params=pltpu.CompilerParams(use_tc_tiling_on_sc=False),
)
def _gather_kernel(idx_hbm, table_hbm, out_hbm, idx_v, rows_v, sem):
    wid = lax.axis_index("s") * _NC + lax.axis_index("c")
    base = wid * _BPW

    def body(i, carry):
        off = base + i * _CHUNK
        pltpu.sync_copy(idx_hbm.at[pl.ds(off, _CHUNK)], idx_v)
        pltpu.async_copy(table_hbm.at[idx_v], rows_v, sem).wait()
        pltpu.sync_copy(rows_v, out_hbm.at[pl.ds(off, _CHUNK)])
        return carry

    lax.fori_loop(0, _NCHUNK, body, 0)


def kernel(inputs, pos_matrix):
    idx = inputs.reshape(-1)
    table = pos_matrix.reshape(_INPUT_DIM, -1)[:, :_OUTPUT_DIM]
    out = _gather_kernel(idx, table)
    return out.reshape(inputs.shape[0], inputs.shape[1], _OUTPUT_DIM)


# speedup vs baseline: 4.9722x; 1.3347x over previous
import functools

import jax
import jax.numpy as jnp
from jax import lax
from jax.experimental import pallas as pl
from jax.experimental.pallas import tpu as pltpu
from jax.experimental.pallas import tpu_sc as plsc

_INPUT_DIM = 8192
_OUTPUT_DIM = 64

_NC = 2
_NS = 16
_NW = _NC * _NS

_B = 4096 * 200
_BPW = _B // _NW
_CHUNK = 128
_NCHUNK = _BPW // _CHUNK
_NB = 8
_NGROUP = _NCHUNK // _NB

_mesh = plsc.VectorSubcoreMesh(core_axis_name="c", subcore_axis_name="s")


@functools.partial(
    pl.kernel,
    out_type=jax.ShapeDtypeStruct((_B, _OUTPUT_DIM), jnp.float32),
    mesh=_mesh,
    scratch_types=[
        pltpu.VMEM((_NCHUNK, _CHUNK), jnp.int32),
        pltpu.VMEM((_NB, _CHUNK, _OUTPUT_DIM), jnp.float32),
        [pltpu.SemaphoreType.DMA] * _NB,
        [pltpu.SemaphoreType.DMA] * _NB,
    ],
    compiler_params=pltpu.CompilerParams(use_tc_tiling_on_sc=False),
)
def _gather_kernel(idx_hbm, table_hbm, out_hbm, idx_v, rows_v, gsems, ssems):
    wid = lax.axis_index("s") * _NC + lax.axis_index("c")
    base = wid * _BPW

    pltpu.sync_copy(idx_hbm.at[wid], idx_v)

    def group(g, carry):
        j0 = g * _NB
        gh = []
        for b in range(_NB):
            gh.append(
                pltpu.async_copy(
                    table_hbm.at[idx_v.at[j0 + b]], rows_v.at[b], gsems[b]
                )
            )
        sh = []
        for b in range(_NB):
            gh[b].wait()
            off = base + (j0 + b) * _CHUNK
            sh.append(
                pltpu.async_copy(
                    rows_v.at[b], out_hbm.at[pl.ds(off, _CHUNK)], ssems[b]
                )
            )
        for b in range(_NB):
            sh[b].wait()
        return carry

    lax.fori_loop(0, _NGROUP, group, 0)


def kernel(inputs, pos_matrix):
    idx = inputs.reshape(_NW, _NCHUNK, _CHUNK)
    table = pos_matrix.reshape(_INPUT_DIM, -1)[:, :_OUTPUT_DIM]
    out = _gather_kernel(idx, table)
    return out.reshape(inputs.shape[0], inputs.shape[1], _OUTPUT_DIM)
